# baseline (speedup 1.0000x reference)
import functools

import jax
import jax.numpy as jnp
import numpy as np
from jax import lax
from jax.experimental import pallas as pl
from jax.experimental.pallas import tpu as pltpu
from jax.experimental.pallas import tpu_sc as plsc

_PAD = 0
_CANDIDATE_MULTIPLE = 2

_NC = 2
_NS = 16
_L = 16

_NEG_INF = np.float32(-np.inf)
_I32_MAX = np.int32(2**31 - 1)
_I32_MIN = np.int32(-(2**31))


_GATHER_DNUMS = lax.GatherDimensionNumbers(
    offset_dims=(), collapsed_slice_dims=(0,), start_index_map=(0,))


def _perm(x, idx):
    return lax.gather(x, idx[:, None], _GATHER_DNUMS, slice_sizes=(1,),
                      mode=lax.GatherScatterMode.PROMISE_IN_BOUNDS)


def _bcast_reduce(x, lanes, op):
    for s in (8, 4, 2, 1):
        x = op(x, _perm(x, lanes ^ s))
    return x


def _insert(x, ix, vs, idxs):
    vs = list(vs)
    idxs = list(idxs)
    for kk in range(len(vs)):
        sel = x > vs[kk]
        nv = jnp.where(sel, x, vs[kk])
        nx = jnp.where(sel, vs[kk], x)
        ni = jnp.where(sel, ix, idxs[kk])
        nix = jnp.where(sel, idxs[kk], ix)
        vs[kk], x = nv, nx
        idxs[kk], ix = ni, nix
    return tuple(vs), tuple(idxs)


def _make_sc_topk(bsz, beam, vocab, k, chunk, group):
    total = beam * vocab
    n_chunks = total // chunk
    n_vregs = chunk // _L
    n_groups = n_vregs // group
    assert total % chunk == 0 and chunk % _L == 0 and n_vregs % group == 0
    assert bsz == _NC * _NS

    mesh = plsc.VectorSubcoreMesh(
        core_axis_name="c", subcore_axis_name="s",
        num_cores=_NC, num_subcores=_NS)

    def body(lp_hbm, bias_hbm, mask_hbm, val_hbm, idx_hbm, beam_hbm,
             buf0, buf1, bias_v, mask_v, res_val, res_idx, res_beam,
             vs_ref, ix_ref, th_ref, sem0, sem1):
        lanes = lax.broadcasted_iota(jnp.int32, (_L,), 0)
        wid = lax.axis_index("s") * _NC + lax.axis_index("c")
        b = wid

        pltpu.sync_copy(bias_hbm.at[pl.ds(b * _L, _L)], bias_v)
        pltpu.sync_copy(mask_hbm.at[pl.ds(b * _L, _L)], mask_v)
        bias_vec = bias_v[...]
        mask_vec = mask_v[...]

        bufs = (buf0, buf1)
        sems = (sem0, sem1)

        def chunk_src(g):
            return lp_hbm.at[pl.ds(b * total + g * chunk, chunk)]

        pltpu.async_copy(chunk_src(0), buf0, sem0)

        def load_lists():
            vs = [vs_ref[pl.ds(kk * _L, _L)] for kk in range(k)]
            idxs = [ix_ref[pl.ds(kk * _L, _L)] for kk in range(k)]
            return vs, idxs

        def store_lists(vs, idxs):
            for kk in range(k):
                vs_ref[pl.ds(kk * _L, _L)] = vs[kk]
                ix_ref[pl.ds(kk * _L, _L)] = idxs[kk]

        store_lists([jnp.full((_L,), _NEG_INF) for _ in range(k)],
                    [jnp.zeros((_L,), jnp.int32) for _ in range(k)])

        def process(buf, g):
            beam_g = (g * chunk) // vocab
            bias_s = _bcast_reduce(
                jnp.where(lanes == beam_g, bias_vec, _NEG_INF),
                lanes, jnp.maximum)
            base = g * chunk

            def thresh(v7):
                t = v7 - bias_s
                return t - (jnp.abs(t) + jnp.abs(v7)) * np.float32(1e-6)

            th_ref[...] = thresh(vs_ref[pl.ds((k - 1) * _L, _L)])

            def group_body(t, _):
                xs = [buf[pl.ds(t * (group * _L) + u * _L, _L)]
                      for u in range(group)]
                th = th_ref[...]
                hit = xs[0] > th
                for u in range(1, group):
                    hit = hit | (xs[u] > th)

                @pl.when(jnp.any(hit))
                def _():
                    vs, idxs = load_lists()
                    for u in range(group):
                        ix = base + t * (group * _L) + u * _L + lanes
                        vs, idxs = _insert(xs[u] + bias_s, ix, vs, idxs)
                    store_lists(vs, idxs)
                    th_ref[...] = thresh(vs[k - 1])

                return 0

            lax.fori_loop(0, n_groups, group_body, 0)

        def two_chunks(i, _):
            for phase in range(2):
                g = i * 2 + phase
                this_buf, this_sem = bufs[phase], sems[phase]
                other_buf, other_sem = bufs[1 - phase], sems[1 - phase]
                pltpu.make_async_copy(chunk_src(0), this_buf, this_sem).wait()

                @pl.when(g + 1 < n_chunks)
                def _():
                    pltpu.async_copy(chunk_src(g + 1), other_buf, other_sem)

                process(this_buf, g)
            return 0

        lax.fori_loop(0, n_chunks // 2, two_chunks, 0)

        vs, idxs = load_lists()
        rv = jnp.full((_L,), _NEG_INF)
        ri = jnp.zeros((_L,), jnp.int32)
        rb = jnp.zeros((_L,), jnp.int32)
        for j in range(k):
            gmax = _bcast_reduce(vs[0], lanes, jnp.maximum)
            gidx = _bcast_reduce(
                jnp.where(vs[0] == gmax, idxs[0], _I32_MAX),
                lanes, jnp.minimum)
            pop = (vs[0] == gmax) & (idxs[0] == gidx)
            beam_s = gidx // vocab
            vocab_s = gidx % vocab
            mval = _bcast_reduce(
                jnp.where(lanes == beam_s, mask_vec, _I32_MIN),
                lanes, jnp.maximum)
            vocab_s = jnp.where(mval == 0, np.int32(_PAD), vocab_s)
            rv = jnp.where(lanes == j, gmax, rv)
            ri = jnp.where(lanes == j, vocab_s, ri)
            rb = jnp.where(lanes == j, beam_s, rb)
            for kk in range(k - 1):
                vs[kk] = jnp.where(pop, vs[kk + 1], vs[kk])
                idxs[kk] = jnp.where(pop, idxs[kk + 1], idxs[kk])
            vs[k - 1] = jnp.where(pop, _NEG_INF, vs[k - 1])
            idxs[k - 1] = jnp.where(pop, np.int32(0), idxs[k - 1])

        res_val[...] = rv
        res_idx[...] = ri
        res_beam[...] = rb
        pltpu.sync_copy(res_val, val_hbm.at[pl.ds(b * _L, _L)])
        pltpu.sync_copy(res_idx, idx_hbm.at[pl.ds(b * _L, _L)])
        pltpu.sync_copy(res_beam, beam_hbm.at[pl.ds(b * _L, _L)])

    return pl.kernel(
        body,
        out_type=(
            jax.ShapeDtypeStruct((bsz * _L,), jnp.float32),
            jax.ShapeDtypeStruct((bsz * _L,), jnp.int32),
            jax.ShapeDtypeStruct((bsz * _L,), jnp.int32),
        ),
        mesh=mesh,
        compiler_params=pltpu.CompilerParams(needs_layout_passes=False),
        scratch_types=[
            pltpu.VMEM((chunk,), jnp.float32),
            pltpu.VMEM((chunk,), jnp.float32),
            pltpu.VMEM((_L,), jnp.float32),
            pltpu.VMEM((_L,), jnp.int32),
            pltpu.VMEM((_L,), jnp.float32),
            pltpu.VMEM((_L,), jnp.int32),
            pltpu.VMEM((_L,), jnp.int32),
            pltpu.VMEM((k * _L,), jnp.float32),
            pltpu.VMEM((k * _L,), jnp.int32),
            pltpu.VMEM((_L,), jnp.float32),
            pltpu.SemaphoreType.DMA,
            pltpu.SemaphoreType.DMA,
        ],
    )


def kernel(step, lprobs, scores, mask_stop_search):
    bsz, beam, vocab = lprobs.shape
    k = _CANDIDATE_MULTIPLE * beam

    step = jnp.asarray(step, jnp.int32)
    bias_later = jnp.take(scores, step, axis=2)
    bias_step0 = jnp.where(jnp.arange(beam) == 0, 0.0, -jnp.inf)[None, :]
    bias = jnp.where(step == 0, bias_step0, bias_later).astype(jnp.float32)

    bias_p = jnp.pad(bias, ((0, 0), (0, _L - beam)),
                     constant_values=-jnp.inf).reshape(-1)
    mask_p = jnp.pad(mask_stop_search.astype(jnp.int32),
                     ((0, 0), (0, _L - beam))).reshape(-1)
    lp_flat = lprobs.reshape(-1)

    topk = _make_sc_topk(bsz, beam, vocab, k, chunk=20000, group=10)
    vals, inds, beams = topk(lp_flat, bias_p, mask_p)
    return (vals.reshape(bsz, _L)[:, :k],
            inds.reshape(bsz, _L)[:, :k],
            beams.reshape(bsz, _L)[:, :k])


# speedup vs baseline: 1.0741x; 1.0741x over previous
import functools

import jax
import jax.numpy as jnp
import numpy as np
from jax import lax
from jax.experimental import pallas as pl
from jax.experimental.pallas import tpu as pltpu
from jax.experimental.pallas import tpu_sc as plsc

_PAD = 0
_CANDIDATE_MULTIPLE = 2

_NC = 2
_NS = 16
_L = 16

_NEG_INF = np.float32(-np.inf)
_I32_MAX = np.int32(2**31 - 1)
_I32_MIN = np.int32(-(2**31))


_GATHER_DNUMS = lax.GatherDimensionNumbers(
    offset_dims=(), collapsed_slice_dims=(0,), start_index_map=(0,))


def _perm(x, idx):
    return lax.gather(x, idx[:, None], _GATHER_DNUMS, slice_sizes=(1,),
                      mode=lax.GatherScatterMode.PROMISE_IN_BOUNDS)


def _bcast_reduce(x, lanes, op):
    for s in (8, 4, 2, 1):
        x = op(x, _perm(x, lanes ^ s))
    return x


def _insert(x, ix, vs, idxs):
    vs = list(vs)
    idxs = list(idxs)
    for kk in range(len(vs)):
        sel = x > vs[kk]
        nv = jnp.where(sel, x, vs[kk])
        nx = jnp.where(sel, vs[kk], x)
        ni = jnp.where(sel, ix, idxs[kk])
        nix = jnp.where(sel, idxs[kk], ix)
        vs[kk], x = nv, nx
        idxs[kk], ix = ni, nix
    return tuple(vs), tuple(idxs)


def _make_sc_topk(bsz, beam, vocab, k, chunk, group, sub):
    total = beam * vocab
    n_chunks = total // chunk
    n_vregs = chunk // _L
    n_groups = n_vregs // group
    assert total % chunk == 0 and chunk % _L == 0 and n_vregs % group == 0
    assert group % sub == 0
    assert bsz == _NC * _NS

    mesh = plsc.VectorSubcoreMesh(
        core_axis_name="c", subcore_axis_name="s",
        num_cores=_NC, num_subcores=_NS)

    def body(lp_hbm, bias_hbm, mask_hbm, val_hbm, idx_hbm, beam_hbm,
             buf0, buf1, bias_v, mask_v, res_val, res_idx, res_beam,
             vs_ref, ix_ref, th_ref, sem0, sem1):
        lanes = lax.broadcasted_iota(jnp.int32, (_L,), 0)
        wid = lax.axis_index("s") * _NC + lax.axis_index("c")
        b = wid

        pltpu.sync_copy(bias_hbm.at[pl.ds(b * _L, _L)], bias_v)
        pltpu.sync_copy(mask_hbm.at[pl.ds(b * _L, _L)], mask_v)
        bias_vec = bias_v[...]
        mask_vec = mask_v[...]

        bufs = (buf0, buf1)
        sems = (sem0, sem1)

        def chunk_src(g):
            return lp_hbm.at[pl.ds(b * total + g * chunk, chunk)]

        pltpu.async_copy(chunk_src(0), buf0, sem0)

        def load_lists():
            vs = [vs_ref[pl.ds(kk * _L, _L)] for kk in range(k)]
            idxs = [ix_ref[pl.ds(kk * _L, _L)] for kk in range(k)]
            return vs, idxs

        def store_lists(vs, idxs):
            for kk in range(k):
                vs_ref[pl.ds(kk * _L, _L)] = vs[kk]
                ix_ref[pl.ds(kk * _L, _L)] = idxs[kk]

        store_lists([jnp.full((_L,), _NEG_INF) for _ in range(k)],
                    [jnp.zeros((_L,), jnp.int32) for _ in range(k)])

        def process(buf, g):
            beam_g = (g * chunk) // vocab
            bias_s = _bcast_reduce(
                jnp.where(lanes == beam_g, bias_vec, _NEG_INF),
                lanes, jnp.maximum)
            base = g * chunk

            def thresh(v7):
                t = v7 - bias_s
                return t - (jnp.abs(t) + jnp.abs(v7)) * np.float32(1e-6)

            th_ref[...] = thresh(vs_ref[pl.ds((k - 1) * _L, _L)])

            def any_lane(pred):
                return plsc.all_reduce_population_count(pred)[0] > 0

            def group_body(t, _):
                xs = [buf[pl.ds(t * (group * _L) + u * _L, _L)]
                      for u in range(group)]
                subs = []
                for s in range(group // sub):
                    m = xs[s * sub]
                    for u in range(s * sub + 1, (s + 1) * sub):
                        m = jnp.maximum(m, xs[u])
                    subs.append(m)
                gm = subs[0]
                for s in range(1, len(subs)):
                    gm = jnp.maximum(gm, subs[s])

                @pl.when(any_lane(gm > th_ref[...]))
                def _():
                    for s in range(group // sub):
                        @pl.when(any_lane(subs[s] > th_ref[...]))
                        def _(s=s):
                            vs, idxs = load_lists()
                            for u in range(s * sub, (s + 1) * sub):
                                ix = (base + t * (group * _L) + u * _L
                                      + lanes)
                                vs, idxs = _insert(xs[u] + bias_s, ix,
                                                   vs, idxs)
                            store_lists(vs, idxs)
                            th_ref[...] = thresh(vs[k - 1])

                return 0

            lax.fori_loop(0, n_groups, group_body, 0)

        def two_chunks(i, _):
            for phase in range(2):
                g = i * 2 + phase
                this_buf, this_sem = bufs[phase], sems[phase]
                other_buf, other_sem = bufs[1 - phase], sems[1 - phase]
                pltpu.make_async_copy(chunk_src(0), this_buf, this_sem).wait()

                @pl.when(g + 1 < n_chunks)
                def _():
                    pltpu.async_copy(chunk_src(g + 1), other_buf, other_sem)

                process(this_buf, g)
            return 0

        lax.fori_loop(0, n_chunks // 2, two_chunks, 0)

        vs, idxs = load_lists()
        rv = jnp.full((_L,), _NEG_INF)
        ri = jnp.zeros((_L,), jnp.int32)
        rb = jnp.zeros((_L,), jnp.int32)
        for j in range(k):
            gmax = _bcast_reduce(vs[0], lanes, jnp.maximum)
            gidx = _bcast_reduce(
                jnp.where(vs[0] == gmax, idxs[0], _I32_MAX),
                lanes, jnp.minimum)
            pop = (vs[0] == gmax) & (idxs[0] == gidx)
            beam_s = gidx // vocab
            vocab_s = gidx % vocab
            mval = _bcast_reduce(
                jnp.where(lanes == beam_s, mask_vec, _I32_MIN),
                lanes, jnp.maximum)
            vocab_s = jnp.where(mval == 0, np.int32(_PAD), vocab_s)
            rv = jnp.where(lanes == j, gmax, rv)
            ri = jnp.where(lanes == j, vocab_s, ri)
            rb = jnp.where(lanes == j, beam_s, rb)
            for kk in range(k - 1):
                vs[kk] = jnp.where(pop, vs[kk + 1], vs[kk])
                idxs[kk] = jnp.where(pop, idxs[kk + 1], idxs[kk])
            vs[k - 1] = jnp.where(pop, _NEG_INF, vs[k - 1])
            idxs[k - 1] = jnp.where(pop, np.int32(0), idxs[k - 1])

        res_val[...] = rv
        res_idx[...] = ri
        res_beam[...] = rb
        pltpu.sync_copy(res_val, val_hbm.at[pl.ds(b * _L, _L)])
        pltpu.sync_copy(res_idx, idx_hbm.at[pl.ds(b * _L, _L)])
        pltpu.sync_copy(res_beam, beam_hbm.at[pl.ds(b * _L, _L)])

    return pl.kernel(
        body,
        out_type=(
            jax.ShapeDtypeStruct((bsz * _L,), jnp.float32),
            jax.ShapeDtypeStruct((bsz * _L,), jnp.int32),
            jax.ShapeDtypeStruct((bsz * _L,), jnp.int32),
        ),
        mesh=mesh,
        compiler_params=pltpu.CompilerParams(
            needs_layout_passes=False,
            disable_bounds_checks=True,
            disable_semaphore_checks=True,
        ),
        scratch_types=[
            pltpu.VMEM((chunk,), jnp.float32),
            pltpu.VMEM((chunk,), jnp.float32),
            pltpu.VMEM((_L,), jnp.float32),
            pltpu.VMEM((_L,), jnp.int32),
            pltpu.VMEM((_L,), jnp.float32),
            pltpu.VMEM((_L,), jnp.int32),
            pltpu.VMEM((_L,), jnp.int32),
            pltpu.VMEM((k * _L,), jnp.float32),
            pltpu.VMEM((k * _L,), jnp.int32),
            pltpu.VMEM((_L,), jnp.float32),
            pltpu.SemaphoreType.DMA,
            pltpu.SemaphoreType.DMA,
        ],
    )


def kernel(step, lprobs, scores, mask_stop_search):
    bsz, beam, vocab = lprobs.shape
    k = _CANDIDATE_MULTIPLE * beam

    step = jnp.asarray(step, jnp.int32)
    bias_later = jnp.take(scores, step, axis=2)
    bias_step0 = jnp.where(jnp.arange(beam) == 0, 0.0, -jnp.inf)[None, :]
    bias = jnp.where(step == 0, bias_step0, bias_later).astype(jnp.float32)

    bias_p = jnp.pad(bias, ((0, 0), (0, _L - beam)),
                     constant_values=-jnp.inf).reshape(-1)
    mask_p = jnp.pad(mask_stop_search.astype(jnp.int32),
                     ((0, 0), (0, _L - beam))).reshape(-1)
    lp_flat = lprobs.reshape(-1)

    topk = _make_sc_topk(bsz, beam, vocab, k, chunk=20000, group=25, sub=5)
    vals, inds, beams = topk(lp_flat, bias_p, mask_p)
    return (vals.reshape(bsz, _L)[:, :k],
            inds.reshape(bsz, _L)[:, :k],
            beams.reshape(bsz, _L)[:, :k])
